# baseline (speedup 1.0000x reference)
import jax
import jax.numpy as jnp
from jax.experimental import pallas as pl
from jax.experimental.pallas import tpu as pltpu

_N_COMP = 8
_TB = 4096


def _body(x_ref, w_ref, b_ref, g_ref, mu_ref, std_ref):
    n = _N_COMP
    out = jnp.dot(x_ref[...], w_ref[...], preferred_element_type=jnp.float32)
    out = out + b_ref[...]
    dz = (out.shape[1] // n - 1) // 2

    pis = out[:, 2 * n * dz:2 * n * dz + n]
    m = jnp.max(pis, axis=-1, keepdims=True)
    e = jnp.exp(pis - m)
    probs = e / jnp.sum(e, axis=-1, keepdims=True)

    s = jnp.log(probs + 1e-30) + g_ref[...]
    smax = jnp.max(s, axis=-1, keepdims=True)
    ismax = (s >= smax).astype(jnp.float32)
    li = jax.lax.broadcasted_iota(jnp.int32, (n, n), 0)
    lj = jax.lax.broadcasted_iota(jnp.int32, (n, n), 1)
    ltri = (li <= lj).astype(jnp.float32)
    po = jnp.dot(ismax, ltri, preferred_element_type=jnp.float32)
    kf = jnp.dot((po == 0.0).astype(jnp.float32),
                 jnp.ones((n, 1), jnp.float32),
                 preferred_element_type=jnp.float32)

    lane = jax.lax.broadcasted_iota(
        jnp.int32, (out.shape[0], 2 * n * dz), 1)
    grp = (lane // dz).astype(jnp.float32)
    mask = (grp == kf) | (grp == kf + n)
    masked = jnp.where(mask, out[:, :2 * n * dz], 0.0)
    rj = jax.lax.broadcasted_iota(jnp.int32, (2 * n * dz, 2 * dz), 0)
    rc = jax.lax.broadcasted_iota(jnp.int32, (2 * n * dz, 2 * dz), 1)
    rsel = (rc == (rj // (n * dz)) * dz + rj % dz).astype(jnp.float32)
    mv = jnp.dot(masked, rsel, preferred_element_type=jnp.float32)
    mu = mv[:, :dz]
    var = mv[:, dz:2 * dz]

    std = jnp.sqrt(jax.nn.softplus(var) + 1e-08)
    std = jnp.clip(std, 1e-05, 100000.0)
    mu_ref[...] = mu
    std_ref[...] = std


def kernel(x, W, b):
    B, d_in = x.shape
    d_out = W.shape[1]
    n = _N_COMP
    dz = (d_out // n - 1) // 2
    span = 2 * dz + 1

    W3 = W.reshape(d_in, n, span)
    Wp = jnp.concatenate([
        W3[:, :, :dz].reshape(d_in, n * dz),
        W3[:, :, dz:2 * dz].reshape(d_in, n * dz),
        W3[:, :, 2 * dz],
    ], axis=1)
    b3 = b.reshape(n, span)
    bp = jnp.concatenate([
        b3[:, :dz].reshape(n * dz),
        b3[:, dz:2 * dz].reshape(n * dz),
        b3[:, 2 * dz],
    ]).reshape(1, d_out)

    G = jax.random.gumbel(jax.random.key(42), (B, n), jnp.float32)

    grid = B // _TB
    mu, std = pl.pallas_call(
        _body,
        grid=(grid,),
        in_specs=[
            pl.BlockSpec((_TB, d_in), lambda i: (i, 0)),
            pl.BlockSpec((d_in, d_out), lambda i: (0, 0)),
            pl.BlockSpec((1, d_out), lambda i: (0, 0)),
            pl.BlockSpec((_TB, n), lambda i: (i, 0)),
        ],
        out_specs=[
            pl.BlockSpec((_TB, dz), lambda i: (i, 0)),
            pl.BlockSpec((_TB, dz), lambda i: (i, 0)),
        ],
        out_shape=[
            jax.ShapeDtypeStruct((B, dz), jnp.float32),
            jax.ShapeDtypeStruct((B, dz), jnp.float32),
        ],
        compiler_params=pltpu.CompilerParams(
            dimension_semantics=("parallel",),
            skip_device_barrier=True,
            allow_input_fusion=[False, True, True, False],
        ),
    )(x, Wp, bp, G)
    return (mu, std)


# speedup vs baseline: 1.0717x; 1.0717x over previous
import jax
import jax.numpy as jnp
from jax.experimental import pallas as pl
from jax.experimental.pallas import tpu as pltpu

_N_COMP = 8
_TB = 8192


def _body(x_ref, w_ref, b_ref, g_ref, mu_ref, std_ref):
    n = _N_COMP
    out = jnp.dot(x_ref[...], w_ref[...], preferred_element_type=jnp.float32)
    out = out + b_ref[...]
    dz = (out.shape[1] // n - 1) // 2

    pis = out[:, 2 * n * dz:2 * n * dz + n]
    m = jnp.max(pis, axis=-1, keepdims=True)
    e = jnp.exp(pis - m)
    probs = e / jnp.sum(e, axis=-1, keepdims=True)

    s = jnp.log(probs + 1e-30) + g_ref[...]
    smax = jnp.max(s, axis=-1, keepdims=True)
    idx = jax.lax.broadcasted_iota(jnp.int32, s.shape, 1)
    k = jnp.min(jnp.where(s >= smax, idx, n), axis=-1, keepdims=True)

    lane = jax.lax.broadcasted_iota(jnp.int32, (out.shape[0], n * dz), 1)
    mask = (lane // dz) == k
    mu = jnp.where(mask, out[:, :n * dz], 0.0)
    var = jnp.where(mask, out[:, n * dz:2 * n * dz], 0.0)
    w = n * dz
    while w > dz:
        w //= 2
        mu = mu[:, :w] + mu[:, w:]
        var = var[:, :w] + var[:, w:]

    std = jnp.sqrt(jax.nn.softplus(var) + 1e-08)
    std = jnp.clip(std, 1e-05, 100000.0)
    mu_ref[...] = mu
    std_ref[...] = std


def kernel(x, W, b):
    B, d_in = x.shape
    d_out = W.shape[1]
    n = _N_COMP
    dz = (d_out // n - 1) // 2
    span = 2 * dz + 1

    W3 = W.reshape(d_in, n, span)
    Wp = jnp.concatenate([
        W3[:, :, :dz].reshape(d_in, n * dz),
        W3[:, :, dz:2 * dz].reshape(d_in, n * dz),
        W3[:, :, 2 * dz],
    ], axis=1)
    b3 = b.reshape(n, span)
    bp = jnp.concatenate([
        b3[:, :dz].reshape(n * dz),
        b3[:, dz:2 * dz].reshape(n * dz),
        b3[:, 2 * dz],
    ]).reshape(1, d_out)

    G = jax.random.gumbel(jax.random.key(42), (B, n), jnp.float32)

    grid = B // _TB
    mu, std = pl.pallas_call(
        _body,
        grid=(grid,),
        in_specs=[
            pl.BlockSpec((_TB, d_in), lambda i: (i, 0)),
            pl.BlockSpec((d_in, d_out), lambda i: (0, 0)),
            pl.BlockSpec((1, d_out), lambda i: (0, 0)),
            pl.BlockSpec((_TB, n), lambda i: (i, 0)),
        ],
        out_specs=[
            pl.BlockSpec((_TB, dz), lambda i: (i, 0)),
            pl.BlockSpec((_TB, dz), lambda i: (i, 0)),
        ],
        out_shape=[
            jax.ShapeDtypeStruct((B, dz), jnp.float32),
            jax.ShapeDtypeStruct((B, dz), jnp.float32),
        ],
        compiler_params=pltpu.CompilerParams(
            dimension_semantics=("parallel",),
            skip_device_barrier=True,
            allow_input_fusion=[False, True, True, False],
        ),
    )(x, Wp, bp, G)
    return (mu, std)


# speedup vs baseline: 1.0862x; 1.0135x over previous
import jax
import jax.numpy as jnp
from jax.experimental import pallas as pl
from jax.experimental.pallas import tpu as pltpu

_N_COMP = 8
_TB = 4096


def _body(x_ref, w_ref, b_ref, g_ref, mu_ref, std_ref):
    n = _N_COMP
    out = jnp.dot(x_ref[...], w_ref[...], preferred_element_type=jnp.float32)
    out = out + b_ref[...]
    dz = (out.shape[1] // n - 1) // 2

    pis = out[:, 2 * n * dz:2 * n * dz + n]
    m = jnp.max(pis, axis=-1, keepdims=True)
    e = jnp.exp(pis - m)
    probs = e / jnp.sum(e, axis=-1, keepdims=True)

    s = jnp.log(probs + 1e-30) + g_ref[...]
    smax = jnp.max(s, axis=-1, keepdims=True)
    idx = jax.lax.broadcasted_iota(jnp.int32, s.shape, 1)
    k = jnp.min(jnp.where(s >= smax, idx, n), axis=-1, keepdims=True)

    lane = jax.lax.broadcasted_iota(jnp.int32, (out.shape[0], n * dz), 1)
    mask = (lane // dz) == k
    mu = jnp.where(mask, out[:, :n * dz], 0.0)
    var = jnp.where(mask, out[:, n * dz:2 * n * dz], 0.0)
    w = n * dz
    while w > dz:
        w //= 2
        mu = mu[:, :w] + mu[:, w:]
        var = var[:, :w] + var[:, w:]

    std = jnp.sqrt(jax.nn.softplus(var) + 1e-08)
    std = jnp.clip(std, 1e-05, 100000.0)
    mu_ref[...] = mu
    std_ref[...] = std


def kernel(x, W, b):
    B, d_in = x.shape
    d_out = W.shape[1]
    n = _N_COMP
    dz = (d_out // n - 1) // 2
    span = 2 * dz + 1

    W3 = W.reshape(d_in, n, span)
    Wp = jnp.concatenate([
        W3[:, :, :dz].reshape(d_in, n * dz),
        W3[:, :, dz:2 * dz].reshape(d_in, n * dz),
        W3[:, :, 2 * dz],
    ], axis=1)
    b3 = b.reshape(n, span)
    bp = jnp.concatenate([
        b3[:, :dz].reshape(n * dz),
        b3[:, dz:2 * dz].reshape(n * dz),
        b3[:, 2 * dz],
    ]).reshape(1, d_out)

    G = jax.random.gumbel(jax.random.key(42), (B, n), jnp.float32)

    grid = B // _TB
    mu, std = pl.pallas_call(
        _body,
        grid=(grid,),
        in_specs=[
            pl.BlockSpec((_TB, d_in), lambda i: (i, 0)),
            pl.BlockSpec((d_in, d_out), lambda i: (0, 0)),
            pl.BlockSpec((1, d_out), lambda i: (0, 0)),
            pl.BlockSpec((_TB, n), lambda i: (i, 0)),
        ],
        out_specs=[
            pl.BlockSpec((_TB, dz), lambda i: (i, 0)),
            pl.BlockSpec((_TB, dz), lambda i: (i, 0)),
        ],
        out_shape=[
            jax.ShapeDtypeStruct((B, dz), jnp.float32),
            jax.ShapeDtypeStruct((B, dz), jnp.float32),
        ],
        compiler_params=pltpu.CompilerParams(
            dimension_semantics=("parallel",),
            skip_device_barrier=True,
            allow_input_fusion=[False, True, True, False],
        ),
    )(x, Wp, bp, G)
    return (mu, std)
